# baseline (speedup 1.0000x reference)
import functools

import jax
import jax.numpy as jnp
from jax import lax
from jax.experimental import pallas as pl
from jax.experimental.pallas import tpu as pltpu
from jax.experimental.pallas import tpu_sc as plsc

N_TAGS = 64
BATCH = 16384
HIST = 200
D = N_TAGS

NW = 32
BW = BATCH // NW
NBT = BW // 128
L8 = 8
NOCT = HIST // L8
LANES = 16
NG = BW // LANES
DT = D // 8


@functools.lru_cache(maxsize=None)
def _make_kernel():
    mesh = plsc.VectorSubcoreMesh(core_axis_name="c", subcore_axis_name="s")

    @functools.partial(
        pl.kernel,
        mesh=mesh,
        out_type=jax.ShapeDtypeStruct((HIST, DT, BATCH // 128, 8, 128),
                                      jnp.float32),
        scratch_types=[
            pltpu.VMEM((N_TAGS * D,), jnp.float32),
            pltpu.VMEM((2, NBT, L8, 128), jnp.int32),
            pltpu.VMEM((2, DT, NBT, 8, 128), jnp.float32),
            pltpu.SemaphoreType.DMA,
            pltpu.SemaphoreType.DMA,
            pltpu.SemaphoreType.DMA,
            pltpu.SemaphoreType.DMA,
            pltpu.SemaphoreType.DMA,
        ],
        compiler_params=pltpu.CompilerParams(needs_layout_passes=False),
    )
    def gather_kernel(idx_hbm, table_hbm, out_hbm, table_v, idx_v, out_v,
                      sem_t, sem_i0, sem_i1, sem_o0, sem_o1):
        cid = lax.axis_index("c")
        sid = lax.axis_index("s")
        wid = sid * 2 + cid
        bt0 = wid * NBT

        pltpu.async_copy(table_hbm, table_v, sem_t).wait()

        def idx_copy0(oct_i):
            return pltpu.make_async_copy(
                idx_hbm.at[oct_i, pl.ds(bt0, NBT), :, :],
                idx_v.at[0], sem_i0)

        def idx_copy1(oct_i):
            return pltpu.make_async_copy(
                idx_hbm.at[oct_i, pl.ds(bt0, NBT), :, :],
                idx_v.at[1], sem_i1)

        def store_copy(l, slot, sem):
            return pltpu.make_async_copy(
                out_v.at[slot],
                out_hbm.at[l, :, pl.ds(bt0, NBT), :, :], sem)

        idx_copy0(0).start()
        idx_copy1(1).start()

        def l_body(l, carry):
            oct_i = l // L8
            i = lax.rem(l, L8)
            islot = lax.rem(oct_i, 2)
            bslot = lax.rem(l, 2)

            @pl.when(jnp.logical_and(i == 0, islot == 0))
            def _():
                idx_copy0(oct_i).wait()

            @pl.when(jnp.logical_and(i == 0, islot == 1))
            def _():
                idx_copy1(oct_i).wait()

            @pl.when(jnp.logical_and(l >= 2, bslot == 0))
            def _():
                store_copy(l - 2, 0, sem_o0).wait()

            @pl.when(jnp.logical_and(l >= 2, bslot == 1))
            def _():
                store_copy(l - 2, 1, sem_o1).wait()

            def gather_group(g, carry2):
                btl = g // L8
                c0 = lax.rem(g, L8) * LANES
                idx16 = idx_v[islot, btl, i, pl.ds(c0, LANES)]
                scaled = idx16 * D
                for d in range(D):
                    vals = plsc.load_gather(table_v, [scaled + d])
                    out_v[bslot, d // 8, btl, d % 8, pl.ds(c0, LANES)] = vals
                return carry2

            lax.fori_loop(0, NG, gather_group, 0)

            @pl.when(bslot == 0)
            def _():
                store_copy(l, 0, sem_o0).start()

            @pl.when(bslot == 1)
            def _():
                store_copy(l, 1, sem_o1).start()

            @pl.when(jnp.logical_and(i == L8 - 1, oct_i + 2 < NOCT))
            def _():
                @pl.when(islot == 0)
                def _():
                    idx_copy0(oct_i + 2).start()

                @pl.when(islot == 1)
                def _():
                    idx_copy1(oct_i + 2).start()
            return carry

        lax.fori_loop(0, HIST, l_body, 0)

        store_copy(HIST - 2, 0, sem_o0).wait()
        store_copy(HIST - 1, 1, sem_o1).wait()

    return gather_kernel


def kernel(idx, layer_matrix):
    idx4 = (idx.T.astype(jnp.int32)
            .reshape(NOCT, L8, BATCH // 128, 128)
            .transpose(0, 2, 1, 3))
    table_flat = layer_matrix.reshape(N_TAGS * D)
    out5 = _make_kernel()(idx4, table_flat)
    return (out5.transpose(2, 4, 0, 1, 3)
            .reshape(BATCH, HIST, D))


# speedup vs baseline: 2.8444x; 2.8444x over previous
import functools

import jax
import jax.numpy as jnp
from jax import lax
from jax.experimental import pallas as pl
from jax.experimental.pallas import tpu as pltpu
from jax.experimental.pallas import tpu_sc as plsc

N_TAGS = 64
BATCH = 16384
HIST = 200
D = N_TAGS

NW = 32
BW = BATCH // NW
NBT = BW // 128
L8 = 8
NOCT = HIST // L8
LANES = 16
NG = BW // LANES
DT = D // 8


@functools.lru_cache(maxsize=None)
def _make_kernel():
    mesh = plsc.VectorSubcoreMesh(core_axis_name="c", subcore_axis_name="s")

    @functools.partial(
        pl.kernel,
        mesh=mesh,
        out_type=jax.ShapeDtypeStruct((HIST, DT, BATCH // 128, 8, 128),
                                      jnp.float32),
        scratch_types=[
            pltpu.VMEM((N_TAGS * D,), jnp.float32),
            pltpu.VMEM((2, NBT, L8, 128), jnp.int32),
            pltpu.VMEM((2, DT, NBT, 8, 128), jnp.float32),
            pltpu.SemaphoreType.DMA,
            pltpu.SemaphoreType.DMA,
            pltpu.SemaphoreType.DMA,
            pltpu.SemaphoreType.DMA,
            pltpu.SemaphoreType.DMA,
        ],
        compiler_params=pltpu.CompilerParams(needs_layout_passes=False),
    )
    def gather_kernel(idx_hbm, table_hbm, out_hbm, table_v, idx_v, out_v,
                      sem_t, sem_i0, sem_i1, sem_o0, sem_o1):
        cid = lax.axis_index("c")
        sid = lax.axis_index("s")
        wid = sid * 2 + cid
        bt0 = wid * NBT

        pltpu.async_copy(table_hbm, table_v, sem_t).wait()

        def idx_copy0(oct_i):
            return pltpu.make_async_copy(
                idx_hbm.at[oct_i, pl.ds(bt0, NBT), :, :],
                idx_v.at[0], sem_i0)

        def idx_copy1(oct_i):
            return pltpu.make_async_copy(
                idx_hbm.at[oct_i, pl.ds(bt0, NBT), :, :],
                idx_v.at[1], sem_i1)

        def store_copy(l, slot, sem):
            return pltpu.make_async_copy(
                out_v.at[slot],
                out_hbm.at[l, :, pl.ds(bt0, NBT), :, :], sem)

        idx_copy0(0).start()
        idx_copy1(1).start()

        def l_body(l, carry):
            oct_i = l // L8
            i = lax.rem(l, L8)
            islot = lax.rem(oct_i, 2)
            bslot = lax.rem(l, 2)

            @pl.when(jnp.logical_and(i == 0, islot == 0))
            def _():
                idx_copy0(oct_i).wait()

            @pl.when(jnp.logical_and(i == 0, islot == 1))
            def _():
                idx_copy1(oct_i).wait()

            @pl.when(jnp.logical_and(l >= 2, bslot == 0))
            def _():
                store_copy(l - 2, 0, sem_o0).wait()

            @pl.when(jnp.logical_and(l >= 2, bslot == 1))
            def _():
                store_copy(l - 2, 1, sem_o1).wait()

            def gather_group(g, carry2):
                btl = g // L8
                c0 = lax.rem(g, L8) * LANES
                idx16 = idx_v[islot, btl, i, pl.ds(c0, LANES)]
                for d in range(D):
                    vals = plsc.load_gather(table_v, [idx16 + d * N_TAGS])
                    out_v[bslot, d // 8, btl, d % 8, pl.ds(c0, LANES)] = vals
                return carry2

            lax.fori_loop(0, NG, gather_group, 0)

            @pl.when(bslot == 0)
            def _():
                store_copy(l, 0, sem_o0).start()

            @pl.when(bslot == 1)
            def _():
                store_copy(l, 1, sem_o1).start()

            @pl.when(jnp.logical_and(i == L8 - 1, oct_i + 2 < NOCT))
            def _():
                @pl.when(islot == 0)
                def _():
                    idx_copy0(oct_i + 2).start()

                @pl.when(islot == 1)
                def _():
                    idx_copy1(oct_i + 2).start()
            return carry

        lax.fori_loop(0, HIST, l_body, 0)

        store_copy(HIST - 2, 0, sem_o0).wait()
        store_copy(HIST - 1, 1, sem_o1).wait()

    return gather_kernel


def kernel(idx, layer_matrix):
    idx4 = (idx.T.astype(jnp.int32)
            .reshape(NOCT, L8, BATCH // 128, 128)
            .transpose(0, 2, 1, 3))
    table_flat = layer_matrix.T.reshape(N_TAGS * D)
    out5 = _make_kernel()(idx4, table_flat)
    return (out5.transpose(2, 4, 0, 1, 3)
            .reshape(BATCH, HIST, D))


# speedup vs baseline: 3.2121x; 1.1293x over previous
import functools

import jax
import jax.numpy as jnp
from jax import lax
from jax.experimental import pallas as pl
from jax.experimental.pallas import tpu as pltpu
from jax.experimental.pallas import tpu_sc as plsc

N_TAGS = 64
BATCH = 16384
HIST = 200
D = N_TAGS

NW = 32
BW = BATCH // NW
NBT = BW // 128
HBT = NBT // 2
L8 = 8
NOCT = HIST // L8
LANES = 16
NGH = HBT * 8
NU = HIST * 2
REP = 16


@functools.lru_cache(maxsize=None)
def _make_kernel():
    mesh = plsc.VectorSubcoreMesh(core_axis_name="c", subcore_axis_name="s")

    @functools.partial(
        pl.kernel,
        mesh=mesh,
        out_type=jax.ShapeDtypeStruct((HIST, D // 8, BATCH // 128, 8, 128),
                                      jnp.float32),
        scratch_types=[
            pltpu.VMEM((N_TAGS * D * REP,), jnp.float32),
            pltpu.VMEM((2, NBT, L8, 128), jnp.int32),
            pltpu.VMEM((2, D // 8, HBT, 8, 128), jnp.float32),
            pltpu.SemaphoreType.DMA,
            pltpu.SemaphoreType.DMA,
            pltpu.SemaphoreType.DMA,
            pltpu.SemaphoreType.DMA,
            pltpu.SemaphoreType.DMA,
        ],
        compiler_params=pltpu.CompilerParams(needs_layout_passes=False),
    )
    def gather_kernel(idx_hbm, table_hbm, out_hbm, table_v, idx_v, out_v,
                      sem_t, sem_i0, sem_i1, sem_o0, sem_o1):
        cid = lax.axis_index("c")
        sid = lax.axis_index("s")
        wid = sid * 2 + cid
        bt0 = wid * NBT

        pltpu.async_copy(table_hbm, table_v, sem_t).wait()
        lane = lax.iota(jnp.int32, LANES)

        def idx_copy0(oct_i):
            return pltpu.make_async_copy(
                idx_hbm.at[oct_i, pl.ds(bt0, NBT), :, :],
                idx_v.at[0], sem_i0)

        def idx_copy1(oct_i):
            return pltpu.make_async_copy(
                idx_hbm.at[oct_i, pl.ds(bt0, NBT), :, :],
                idx_v.at[1], sem_i1)

        def store_copy(u, slot, sem):
            return pltpu.make_async_copy(
                out_v.at[slot],
                out_hbm.at[u // 2, :, pl.ds(bt0 + lax.rem(u, 2) * HBT, HBT),
                           :, :], sem)

        idx_copy0(0).start()
        idx_copy1(1).start()

        def u_body(u, carry):
            l = u // 2
            h = lax.rem(u, 2)
            oct_i = l // L8
            i = lax.rem(l, L8)
            islot = lax.rem(oct_i, 2)
            uslot = lax.rem(u, 2)

            @pl.when(jnp.logical_and(lax.rem(u, 16) == 0, islot == 0))
            def _():
                idx_copy0(oct_i).wait()

            @pl.when(jnp.logical_and(lax.rem(u, 16) == 0, islot == 1))
            def _():
                idx_copy1(oct_i).wait()

            @pl.when(jnp.logical_and(u >= 2, uslot == 0))
            def _():
                store_copy(u - 2, 0, sem_o0).wait()

            @pl.when(jnp.logical_and(u >= 2, uslot == 1))
            def _():
                store_copy(u - 2, 1, sem_o1).wait()

            def gather_group(g, carry2):
                btl = h * HBT + g // 8
                c0 = lax.rem(g, 8) * LANES
                idx16 = idx_v[islot, btl, i, pl.ds(c0, LANES)]
                pre = idx16 * REP + lane
                for d in range(D):
                    vals = plsc.load_gather(table_v, [pre + d * (N_TAGS * REP)])
                    out_v[uslot, d // 8, g // 8, d % 8, pl.ds(c0, LANES)] = vals
                return carry2

            lax.fori_loop(0, NGH, gather_group, 0)

            @pl.when(uslot == 0)
            def _():
                store_copy(u, 0, sem_o0).start()

            @pl.when(uslot == 1)
            def _():
                store_copy(u, 1, sem_o1).start()

            @pl.when(jnp.logical_and(lax.rem(u, 16) == 15, oct_i + 2 < NOCT))
            def _():
                @pl.when(islot == 0)
                def _():
                    idx_copy0(oct_i + 2).start()

                @pl.when(islot == 1)
                def _():
                    idx_copy1(oct_i + 2).start()
            return carry

        lax.fori_loop(0, NU, u_body, 0)

        store_copy(NU - 2, 0, sem_o0).wait()
        store_copy(NU - 1, 1, sem_o1).wait()

    return gather_kernel


def kernel(idx, layer_matrix):
    idx4 = (idx.T.astype(jnp.int32)
            .reshape(NOCT, L8, BATCH // 128, 128)
            .transpose(0, 2, 1, 3))
    table_rep = jnp.tile(layer_matrix.T.reshape(N_TAGS * D, 1),
                         (1, REP)).reshape(N_TAGS * D * REP)
    out5 = _make_kernel()(idx4, table_rep)
    return (out5.transpose(2, 4, 0, 1, 3)
            .reshape(BATCH, HIST, D))


# speedup vs baseline: 5.6414x; 1.7563x over previous
import functools

import jax
import jax.numpy as jnp
from jax import lax
from jax.experimental import pallas as pl
from jax.experimental.pallas import tpu as pltpu
from jax.experimental.pallas import tpu_sc as plsc

N_TAGS = 64
BATCH = 16384
HIST = 200
D = N_TAGS

NW = 32
BW = BATCH // NW
NBT = BW // 128
HBT = NBT // 2
L8 = 8
NOCT = HIST // L8
LANES = 16
NGH = HBT * 8
NU = HIST * 2
REP = 16


@functools.lru_cache(maxsize=None)
def _make_kernel():
    mesh = plsc.VectorSubcoreMesh(core_axis_name="c", subcore_axis_name="s")

    @functools.partial(
        pl.kernel,
        mesh=mesh,
        out_type=jax.ShapeDtypeStruct((HIST, D // 8, BATCH // 128, 8, 128),
                                      jnp.float32),
        scratch_types=[
            pltpu.VMEM((N_TAGS * D * REP,), jnp.float32),
            pltpu.VMEM((2, NBT, L8, 128), jnp.int32),
            pltpu.VMEM((2, D // 8, HBT, 8, 128), jnp.float32),
            pltpu.SemaphoreType.DMA,
            pltpu.SemaphoreType.DMA,
            pltpu.SemaphoreType.DMA,
            pltpu.SemaphoreType.DMA,
            pltpu.SemaphoreType.DMA,
        ],
        compiler_params=pltpu.CompilerParams(needs_layout_passes=False),
    )
    def gather_kernel(idx_hbm, table_hbm, out_hbm, table_v, idx_v, out_v,
                      sem_t, sem_i0, sem_i1, sem_o0, sem_o1):
        cid = lax.axis_index("c")
        sid = lax.axis_index("s")
        wid = sid * 2 + cid
        bt0 = wid * NBT

        pltpu.async_copy(table_hbm, table_v, sem_t).wait()
        lane = lax.iota(jnp.int32, LANES)

        def idx_copy0(oct_i):
            return pltpu.make_async_copy(
                idx_hbm.at[oct_i, pl.ds(bt0, NBT), :, :],
                idx_v.at[0], sem_i0)

        def idx_copy1(oct_i):
            return pltpu.make_async_copy(
                idx_hbm.at[oct_i, pl.ds(bt0, NBT), :, :],
                idx_v.at[1], sem_i1)

        def store_copy(u, slot, sem):
            return pltpu.make_async_copy(
                out_v.at[slot],
                out_hbm.at[u // 2, :, pl.ds(bt0 + lax.rem(u, 2) * HBT, HBT),
                           :, :], sem)

        idx_copy0(0).start()
        idx_copy1(1).start()

        def u_body(u, carry):
            l = u // 2
            h = lax.rem(u, 2)
            oct_i = l // L8
            i = lax.rem(l, L8)
            islot = lax.rem(oct_i, 2)
            uslot = lax.rem(u, 2)

            @pl.when(jnp.logical_and(lax.rem(u, 16) == 0, islot == 0))
            def _():
                idx_copy0(oct_i).wait()

            @pl.when(jnp.logical_and(lax.rem(u, 16) == 0, islot == 1))
            def _():
                idx_copy1(oct_i).wait()

            @pl.when(jnp.logical_and(u >= 2, uslot == 0))
            def _():
                store_copy(u - 2, 0, sem_o0).wait()

            @pl.when(jnp.logical_and(u >= 2, uslot == 1))
            def _():
                store_copy(u - 2, 1, sem_o1).wait()

            def gather_pair(gg, carry2):
                ga = 2 * gg
                gb = 2 * gg + 1
                btl = h * HBT + ga // 8
                c0a = lax.rem(ga, 8) * LANES
                c0b = lax.rem(gb, 8) * LANES
                idx_a = idx_v[islot, btl, i, pl.ds(c0a, LANES)]
                idx_b = idx_v[islot, btl, i, pl.ds(c0b, LANES)]
                pre_a = idx_a * REP + lane
                pre_b = idx_b * REP + lane
                for d in range(D):
                    row = table_v.at[pl.ds(d * N_TAGS * REP, N_TAGS * REP)]
                    va = plsc.load_gather(row, [pre_a])
                    vb = plsc.load_gather(row, [pre_b])
                    out_v[uslot, d // 8, ga // 8, d % 8, pl.ds(c0a, LANES)] = va
                    out_v[uslot, d // 8, gb // 8, d % 8, pl.ds(c0b, LANES)] = vb
                return carry2

            lax.fori_loop(0, NGH // 2, gather_pair, 0)

            @pl.when(uslot == 0)
            def _():
                store_copy(u, 0, sem_o0).start()

            @pl.when(uslot == 1)
            def _():
                store_copy(u, 1, sem_o1).start()

            @pl.when(jnp.logical_and(lax.rem(u, 16) == 15, oct_i + 2 < NOCT))
            def _():
                @pl.when(islot == 0)
                def _():
                    idx_copy0(oct_i + 2).start()

                @pl.when(islot == 1)
                def _():
                    idx_copy1(oct_i + 2).start()
            return carry

        lax.fori_loop(0, NU, u_body, 0)

        store_copy(NU - 2, 0, sem_o0).wait()
        store_copy(NU - 1, 1, sem_o1).wait()

    return gather_kernel


def kernel(idx, layer_matrix):
    idx4 = (idx.T.astype(jnp.int32)
            .reshape(NOCT, L8, BATCH // 128, 128)
            .transpose(0, 2, 1, 3))
    table_rep = jnp.tile(layer_matrix.T.reshape(N_TAGS * D, 1),
                         (1, REP)).reshape(N_TAGS * D * REP)
    out5 = _make_kernel()(idx4, table_rep)
    return (out5.transpose(2, 4, 0, 1, 3)
            .reshape(BATCH, HIST, D))


# speedup vs baseline: 8.6028x; 1.5249x over previous
import functools

import jax
import jax.numpy as jnp
from jax import lax
from jax.experimental import pallas as pl
from jax.experimental.pallas import tpu as pltpu
from jax.experimental.pallas import tpu_sc as plsc

N_TAGS = 64
BATCH = 16384
HIST = 200
D = N_TAGS

NW = 32
BW = BATCH // NW
NBT = BW // 128
HBT = NBT // 2
L8 = 8
NOCT = HIST // L8
LANES = 16
NGH = HBT * 8
NU = HIST * 2
REP = 16


@functools.lru_cache(maxsize=None)
def _make_kernel():
    mesh = plsc.VectorSubcoreMesh(core_axis_name="c", subcore_axis_name="s")

    @functools.partial(
        pl.kernel,
        mesh=mesh,
        out_type=jax.ShapeDtypeStruct((HIST, D // 8, BATCH // 128, 8, 128),
                                      jnp.float32),
        scratch_types=[
            pltpu.VMEM((N_TAGS * D * REP,), jnp.float32),
            pltpu.VMEM((2, NBT, L8, 128), jnp.int32),
            pltpu.VMEM((2, D // 8, HBT, 8, 128), jnp.float32),
            pltpu.SemaphoreType.DMA,
            pltpu.SemaphoreType.DMA,
            pltpu.SemaphoreType.DMA,
            pltpu.SemaphoreType.DMA,
            pltpu.SemaphoreType.DMA,
        ],
        compiler_params=pltpu.CompilerParams(needs_layout_passes=False),
    )
    def gather_kernel(idx_hbm, table_hbm, out_hbm, table_v, idx_v, out_v,
                      sem_t, sem_i0, sem_i1, sem_o0, sem_o1):
        cid = lax.axis_index("c")
        sid = lax.axis_index("s")
        wid = sid * 2 + cid
        bt0 = wid * NBT

        pltpu.async_copy(table_hbm, table_v, sem_t).wait()
        lane = lax.iota(jnp.int32, LANES)

        def idx_copy0(oct_i):
            return pltpu.make_async_copy(
                idx_hbm.at[oct_i, pl.ds(bt0, NBT), :, :],
                idx_v.at[0], sem_i0)

        def idx_copy1(oct_i):
            return pltpu.make_async_copy(
                idx_hbm.at[oct_i, pl.ds(bt0, NBT), :, :],
                idx_v.at[1], sem_i1)

        def store_copy(u, slot, sem):
            return pltpu.make_async_copy(
                out_v.at[slot],
                out_hbm.at[u // 2, :, pl.ds(bt0 + lax.rem(u, 2) * HBT, HBT),
                           :, :], sem)

        idx_copy0(0).start()
        idx_copy1(1).start()

        def u_body(u, carry):
            l = u // 2
            h = lax.rem(u, 2)
            oct_i = l // L8
            i = lax.rem(l, L8)
            islot = lax.rem(oct_i, 2)
            uslot = lax.rem(u, 2)

            @pl.when(jnp.logical_and(lax.rem(u, 16) == 0, islot == 0))
            def _():
                idx_copy0(oct_i).wait()

            @pl.when(jnp.logical_and(lax.rem(u, 16) == 0, islot == 1))
            def _():
                idx_copy1(oct_i).wait()

            @pl.when(jnp.logical_and(u >= 2, uslot == 0))
            def _():
                store_copy(u - 2, 0, sem_o0).wait()

            @pl.when(jnp.logical_and(u >= 2, uslot == 1))
            def _():
                store_copy(u - 2, 1, sem_o1).wait()

            def gather_quad(gg, carry2):
                gs = [4 * gg + k for k in range(4)]
                btl = h * HBT + gs[0] // 8
                c0s = [lax.rem(g, 8) * LANES for g in gs]
                pres = [idx_v[islot, btl, i, pl.ds(c0, LANES)] * REP + lane
                        for c0 in c0s]
                for d in range(D):
                    row = table_v.at[pl.ds(d * N_TAGS * REP, N_TAGS * REP)]
                    vals = [plsc.load_gather(row, [pre]) for pre in pres]
                    for k in range(4):
                        out_v[uslot, d // 8, gs[k] // 8, d % 8,
                              pl.ds(c0s[k], LANES)] = vals[k]
                return carry2

            lax.fori_loop(0, NGH // 4, gather_quad, 0)

            @pl.when(uslot == 0)
            def _():
                store_copy(u, 0, sem_o0).start()

            @pl.when(uslot == 1)
            def _():
                store_copy(u, 1, sem_o1).start()

            @pl.when(jnp.logical_and(lax.rem(u, 16) == 15, oct_i + 2 < NOCT))
            def _():
                @pl.when(islot == 0)
                def _():
                    idx_copy0(oct_i + 2).start()

                @pl.when(islot == 1)
                def _():
                    idx_copy1(oct_i + 2).start()
            return carry

        lax.fori_loop(0, NU, u_body, 0)

        store_copy(NU - 2, 0, sem_o0).wait()
        store_copy(NU - 1, 1, sem_o1).wait()

    return gather_kernel


def kernel(idx, layer_matrix):
    idx4 = (idx.T.astype(jnp.int32)
            .reshape(NOCT, L8, BATCH // 128, 128)
            .transpose(0, 2, 1, 3))
    table_rep = jnp.tile(layer_matrix.T.reshape(N_TAGS * D, 1),
                         (1, REP)).reshape(N_TAGS * D * REP)
    out5 = _make_kernel()(idx4, table_rep)
    return (out5.transpose(2, 4, 0, 1, 3)
            .reshape(BATCH, HIST, D))


# speedup vs baseline: 10.2726x; 1.1941x over previous
import functools

import jax
import jax.numpy as jnp
from jax import lax
from jax.experimental import pallas as pl
from jax.experimental.pallas import tpu as pltpu
from jax.experimental.pallas import tpu_sc as plsc

N_TAGS = 64
BATCH = 16384
HIST = 200
D = N_TAGS

NW = 32
BW = BATCH // NW
NBT = BW // 128
HBT = NBT // 2
L8 = 8
NOCT = HIST // L8
LANES = 16
NGH = HBT * 8
NU = HIST * 2
REP = 16


@functools.lru_cache(maxsize=None)
def _make_kernel():
    mesh = plsc.VectorSubcoreMesh(core_axis_name="c", subcore_axis_name="s")

    @functools.partial(
        pl.kernel,
        mesh=mesh,
        out_type=jax.ShapeDtypeStruct((HIST, D // 8, BATCH // 128, 8, 128),
                                      jnp.float32),
        scratch_types=[
            pltpu.VMEM((N_TAGS * D * REP,), jnp.float32),
            pltpu.VMEM((2, NBT, L8, 128), jnp.int32),
            pltpu.VMEM((2, D // 8, HBT, 8, 128), jnp.float32),
            pltpu.SemaphoreType.DMA,
            pltpu.SemaphoreType.DMA,
            pltpu.SemaphoreType.DMA,
            pltpu.SemaphoreType.DMA,
            pltpu.SemaphoreType.DMA,
        ],
        compiler_params=pltpu.CompilerParams(needs_layout_passes=False),
    )
    def gather_kernel(idx_hbm, table_hbm, out_hbm, table_v, idx_v, out_v,
                      sem_t, sem_i0, sem_i1, sem_o0, sem_o1):
        cid = lax.axis_index("c")
        sid = lax.axis_index("s")
        wid = sid * 2 + cid
        bt0 = wid * NBT

        pltpu.async_copy(table_hbm, table_v, sem_t).wait()
        lane = lax.iota(jnp.int32, LANES)

        def idx_copy0(oct_i):
            return pltpu.make_async_copy(
                idx_hbm.at[oct_i, pl.ds(bt0, NBT), :, :],
                idx_v.at[0], sem_i0)

        def idx_copy1(oct_i):
            return pltpu.make_async_copy(
                idx_hbm.at[oct_i, pl.ds(bt0, NBT), :, :],
                idx_v.at[1], sem_i1)

        def store_copy(u, slot, sem):
            return pltpu.make_async_copy(
                out_v.at[slot],
                out_hbm.at[u // 2, :, pl.ds(bt0 + lax.rem(u, 2) * HBT, HBT),
                           :, :], sem)

        idx_copy0(0).start()
        idx_copy1(1).start()

        def u_body(u, carry):
            l = u // 2
            h = lax.rem(u, 2)
            oct_i = l // L8
            i = lax.rem(l, L8)
            islot = lax.rem(oct_i, 2)
            uslot = lax.rem(u, 2)

            @pl.when(jnp.logical_and(lax.rem(u, 16) == 0, islot == 0))
            def _():
                idx_copy0(oct_i).wait()

            @pl.when(jnp.logical_and(lax.rem(u, 16) == 0, islot == 1))
            def _():
                idx_copy1(oct_i).wait()

            @pl.when(jnp.logical_and(u >= 2, uslot == 0))
            def _():
                store_copy(u - 2, 0, sem_o0).wait()

            @pl.when(jnp.logical_and(u >= 2, uslot == 1))
            def _():
                store_copy(u - 2, 1, sem_o1).wait()

            def gather_oct(gg, carry2):
                gs = [8 * gg + k for k in range(8)]
                btl = h * HBT + gs[0] // 8
                c0s = [lax.rem(g, 8) * LANES for g in gs]
                pres = [idx_v[islot, btl, i, pl.ds(c0, LANES)] * REP + lane
                        for c0 in c0s]
                for d in range(D):
                    row = table_v.at[pl.ds(d * N_TAGS * REP, N_TAGS * REP)]
                    vals = [plsc.load_gather(row, [pre]) for pre in pres]
                    for k in range(8):
                        out_v[uslot, d // 8, gs[k] // 8, d % 8,
                              pl.ds(c0s[k], LANES)] = vals[k]
                return carry2

            lax.fori_loop(0, NGH // 8, gather_oct, 0)

            @pl.when(uslot == 0)
            def _():
                store_copy(u, 0, sem_o0).start()

            @pl.when(uslot == 1)
            def _():
                store_copy(u, 1, sem_o1).start()

            @pl.when(jnp.logical_and(lax.rem(u, 16) == 15, oct_i + 2 < NOCT))
            def _():
                @pl.when(islot == 0)
                def _():
                    idx_copy0(oct_i + 2).start()

                @pl.when(islot == 1)
                def _():
                    idx_copy1(oct_i + 2).start()
            return carry

        lax.fori_loop(0, NU, u_body, 0)

        store_copy(NU - 2, 0, sem_o0).wait()
        store_copy(NU - 1, 1, sem_o1).wait()

    return gather_kernel


def kernel(idx, layer_matrix):
    idx4 = (idx.T.astype(jnp.int32)
            .reshape(NOCT, L8, BATCH // 128, 128)
            .transpose(0, 2, 1, 3))
    table_rep = jnp.tile(layer_matrix.T.reshape(N_TAGS * D, 1),
                         (1, REP)).reshape(N_TAGS * D * REP)
    out5 = _make_kernel()(idx4, table_rep)
    return (out5.transpose(2, 4, 0, 1, 3)
            .reshape(BATCH, HIST, D))
